# baseline (speedup 1.0000x reference)
import functools

import jax
import jax.numpy as jnp
from jax import lax
from jax.experimental import pallas as pl
from jax.experimental.pallas import tpu as pltpu
from jax.experimental.pallas import tpu_sc as plsc

NC = 2
NS = 16
NW = NC * NS
LANES = 16

N_STRUCT = 16384
N_ATOMS = 200
N_TOTAL = N_STRUCT * N_ATOMS
PER_W = N_TOTAL // NW
CHUNK = 12800
N_CHUNKS = PER_W // CHUNK
N_VECS = CHUNK // LANES


def _sc_body(values_hbm, idx_hbm, out_hbm, values_v, idx_v, out_v):
    wid = lax.axis_index("s") * NC + lax.axis_index("c")
    base = wid * PER_W

    pltpu.sync_copy(values_hbm, values_v)

    @pl.loop(0, N_CHUNKS)
    def _chunk(c):
        off = base + c * CHUNK
        pltpu.sync_copy(idx_hbm.at[pl.ds(off, CHUNK)], idx_v)

        @plsc.parallel_loop(0, N_VECS, unroll=8)
        def _gather(i):
            iv = idx_v[pl.ds(i * LANES, LANES)]
            out_v[pl.ds(i * LANES, LANES)] = plsc.load_gather(values_v, [iv])

        pltpu.sync_copy(out_v, out_hbm.at[pl.ds(off, CHUNK)])


@jax.jit
def _lookup(values, idx_flat):
    mesh = plsc.VectorSubcoreMesh(
        core_axis_name="c", subcore_axis_name="s", num_cores=NC,
        num_subcores=NS,
    )
    return pl.kernel(
        _sc_body,
        out_type=jax.ShapeDtypeStruct((N_TOTAL,), jnp.float32),
        mesh=mesh,
        scratch_types=[
            pltpu.VMEM((64,), jnp.float32),
            pltpu.VMEM((CHUNK,), jnp.int32),
            pltpu.VMEM((CHUNK,), jnp.float32),
        ],
        compiler_params=pltpu.CompilerParams(needs_layout_passes=False),
    )(values, idx_flat)


def kernel(values, index):
    out = _lookup(values, index.reshape(-1))
    return out.reshape(N_STRUCT, N_ATOMS)


# speedup vs baseline: 1.0868x; 1.0868x over previous
import functools

import jax
import jax.numpy as jnp
from jax import lax
from jax.experimental import pallas as pl
from jax.experimental.pallas import tpu as pltpu
from jax.experimental.pallas import tpu_sc as plsc

NC = 2
NS = 16
NW = NC * NS
LANES = 16

N_STRUCT = 16384
N_ATOMS = 200
N_TOTAL = N_STRUCT * N_ATOMS
PER_W = N_TOTAL // NW
CHUNK = 12800
N_CHUNKS = PER_W // CHUNK
N_VECS = CHUNK // LANES


def _sc_body(values_hbm, idx_hbm, out_hbm, values_v,
             idx0, idx1, out0, out1, sem_i0, sem_i1, sem_o0, sem_o1):
    wid = lax.axis_index("s") * NC + lax.axis_index("c")
    base = wid * PER_W

    pltpu.sync_copy(values_hbm, values_v)

    idx_bufs = (idx0, idx1)
    out_bufs = (out0, out1)
    sem_in = (sem_i0, sem_i1)
    sem_out = (sem_o0, sem_o1)

    in_dma = [None, None]
    out_dma = [None, None]

    in_dma[0] = pltpu.async_copy(
        idx_hbm.at[pl.ds(base, CHUNK)], idx_bufs[0], sem_in[0])

    for c in range(N_CHUNKS):
        b = c % 2
        nb = (c + 1) % 2
        if c + 1 < N_CHUNKS:
            in_dma[nb] = pltpu.async_copy(
                idx_hbm.at[pl.ds(base + (c + 1) * CHUNK, CHUNK)],
                idx_bufs[nb], sem_in[nb])
        in_dma[b].wait()
        if out_dma[b] is not None:
            out_dma[b].wait()

        idx_v = idx_bufs[b]
        out_v = out_bufs[b]

        @plsc.parallel_loop(0, N_VECS, unroll=8)
        def _gather(i):
            iv = idx_v[pl.ds(i * LANES, LANES)]
            out_v[pl.ds(i * LANES, LANES)] = plsc.load_gather(values_v, [iv])

        out_dma[b] = pltpu.async_copy(
            out_v, out_hbm.at[pl.ds(base + c * CHUNK, CHUNK)], sem_out[b])

    for b in range(2):
        if out_dma[b] is not None:
            out_dma[b].wait()


@jax.jit
def _lookup(values, idx_flat):
    mesh = plsc.VectorSubcoreMesh(
        core_axis_name="c", subcore_axis_name="s", num_cores=NC,
        num_subcores=NS,
    )
    return pl.kernel(
        _sc_body,
        out_type=jax.ShapeDtypeStruct((N_TOTAL,), jnp.float32),
        mesh=mesh,
        scratch_types=[
            pltpu.VMEM((64,), jnp.float32),
            pltpu.VMEM((CHUNK,), jnp.int32),
            pltpu.VMEM((CHUNK,), jnp.int32),
            pltpu.VMEM((CHUNK,), jnp.float32),
            pltpu.VMEM((CHUNK,), jnp.float32),
            pltpu.SemaphoreType.DMA,
            pltpu.SemaphoreType.DMA,
            pltpu.SemaphoreType.DMA,
            pltpu.SemaphoreType.DMA,
        ],
        compiler_params=pltpu.CompilerParams(needs_layout_passes=False),
    )(values, idx_flat)


def kernel(values, index):
    out = _lookup(values, index.reshape(-1))
    return out.reshape(N_STRUCT, N_ATOMS)


# speedup vs baseline: 1.8742x; 1.7245x over previous
import jax
import jax.numpy as jnp
from jax import lax
from jax.experimental import pallas as pl
from jax.experimental.pallas import tpu as pltpu
from jax.experimental.pallas import tpu_sc as plsc

NC = 2
NS = 16
NW = NC * NS
LANES = 16

N_STRUCT = 16384
N_ATOMS = 200
ROWS_PER_W = N_STRUCT // NW
CHUNK_ROWS = 64
N_CHUNKS = ROWS_PER_W // CHUNK_ROWS

OFFSETS = list(range(0, 128, 16)) + [128, 144, 160, 176, 184]


def _sc_body(values_hbm, idx_hbm, out_hbm, values_v,
             idx0, idx1, out0, out1, sem_i0, sem_i1, sem_o0, sem_o1):
    wid = lax.axis_index("s") * NC + lax.axis_index("c")
    row_base = wid * ROWS_PER_W

    pltpu.sync_copy(values_hbm, values_v)

    idx_bufs = (idx0, idx1)
    out_bufs = (out0, out1)
    sem_in = (sem_i0, sem_i1)
    sem_out = (sem_o0, sem_o1)

    in_dma = [None, None]
    out_dma = [None, None]

    in_dma[0] = pltpu.async_copy(
        idx_hbm.at[pl.ds(row_base, CHUNK_ROWS)], idx_bufs[0], sem_in[0])

    for c in range(N_CHUNKS):
        b = c % 2
        nb = (c + 1) % 2
        if c + 1 < N_CHUNKS:
            in_dma[nb] = pltpu.async_copy(
                idx_hbm.at[pl.ds(row_base + (c + 1) * CHUNK_ROWS, CHUNK_ROWS)],
                idx_bufs[nb], sem_in[nb])
        in_dma[b].wait()
        if out_dma[b] is not None:
            out_dma[b].wait()

        idx_v = idx_bufs[b]
        out_v = out_bufs[b]

        @plsc.parallel_loop(0, CHUNK_ROWS, unroll=2)
        def _row(r):
            for off in OFFSETS:
                iv = idx_v[r, pl.ds(off, LANES)]
                out_v[r, pl.ds(off, LANES)] = plsc.load_gather(values_v, [iv])

        out_dma[b] = pltpu.async_copy(
            out_v, out_hbm.at[pl.ds(row_base + c * CHUNK_ROWS, CHUNK_ROWS)],
            sem_out[b])

    for b in range(2):
        if out_dma[b] is not None:
            out_dma[b].wait()


@jax.jit
def _lookup(values, index):
    mesh = plsc.VectorSubcoreMesh(
        core_axis_name="c", subcore_axis_name="s", num_cores=NC,
        num_subcores=NS,
    )
    return pl.kernel(
        _sc_body,
        out_type=jax.ShapeDtypeStruct((N_STRUCT, N_ATOMS), jnp.float32),
        mesh=mesh,
        scratch_types=[
            pltpu.VMEM((64,), jnp.float32),
            pltpu.VMEM((CHUNK_ROWS, N_ATOMS), jnp.int32),
            pltpu.VMEM((CHUNK_ROWS, N_ATOMS), jnp.int32),
            pltpu.VMEM((CHUNK_ROWS, N_ATOMS), jnp.float32),
            pltpu.VMEM((CHUNK_ROWS, N_ATOMS), jnp.float32),
            pltpu.SemaphoreType.DMA,
            pltpu.SemaphoreType.DMA,
            pltpu.SemaphoreType.DMA,
            pltpu.SemaphoreType.DMA,
        ],
        compiler_params=pltpu.CompilerParams(
            needs_layout_passes=False,
            use_tc_tiling_on_sc=True,
        ),
    )(values, index)


def kernel(values, index):
    return _lookup(values, index)
